# baseline (speedup 1.0000x reference)
import functools

import jax
import jax.numpy as jnp
from jax import lax
from jax.experimental import pallas as pl
from jax.experimental.pallas import tpu as pltpu
from jax.experimental.pallas import tpu_sc as plsc

_INFO = plsc.get_sparse_core_info()
_NC = _INFO.num_cores
_NS = _INFO.num_subcores
_NW = _NC * _NS

_CB = 4
_NBUF = 2


def _make_gather(batch: int, seq: int, dim: int):
    assert batch % (_NW * _CB * _NBUF) == 0
    rows_per_w = batch // _NW
    n_groups = rows_per_w // (_CB * _NBUF)
    chunk = _CB * seq
    mesh = plsc.VectorSubcoreMesh(core_axis_name="c", subcore_axis_name="s")

    @functools.partial(
        pl.kernel,
        mesh=mesh,
        out_type=jax.ShapeDtypeStruct((batch * seq, 2 * dim), jnp.float32),
        scratch_types=[
            [pltpu.VMEM((_CB, seq), jnp.int32) for _ in range(_NBUF)],
            [pltpu.VMEM((chunk, dim), jnp.float32) for _ in range(_NBUF)],
            [pltpu.SemaphoreType.DMA for _ in range(_NBUF)],
            [pltpu.SemaphoreType.DMA for _ in range(_NBUF)],
        ],
        compiler_params=pltpu.CompilerParams(use_tc_tiling_on_sc=False),
    )
    def gather_kernel(idx_hbm, table_hbm, out_hbm, idx_v, rows, gsems, wsems):
        wid = lax.axis_index("s") * _NC + lax.axis_index("c")
        base_row = wid * rows_per_w

        def group(g, carry):
            gathers = []
            for b in range(_NBUF):
                r0 = base_row + (g * _NBUF + b) * _CB

                @pl.when(g > 0)
                def _(b=b, r0=r0):
                    pltpu.make_async_copy(
                        rows[b],
                        out_hbm.at[pl.ds(r0 * seq, chunk), pl.ds(0, dim)],
                        wsems[b],
                    ).wait()

                pltpu.sync_copy(idx_hbm.at[pl.ds(r0, _CB)], idx_v[b])
                for j in range(_CB):
                    gathers.append(
                        pltpu.async_copy(
                            table_hbm.at[idx_v[b].at[j]],
                            rows[b].at[pl.ds(j * seq, seq)],
                            gsems[b],
                        )
                    )
            for b in range(_NBUF):
                r0 = base_row + (g * _NBUF + b) * _CB
                for j in range(_CB):
                    gathers[b * _CB + j].wait()
                pltpu.async_copy(
                    rows[b],
                    out_hbm.at[pl.ds(r0 * seq, chunk), pl.ds(0, dim)],
                    wsems[b],
                )
            return carry

        lax.fori_loop(0, n_groups, group, 0)

        for b in range(_NBUF):
            r0 = base_row + ((n_groups - 1) * _NBUF + b) * _CB
            pltpu.make_async_copy(
                rows[b],
                out_hbm.at[pl.ds(r0 * seq, chunk), pl.ds(0, dim)],
                wsems[b],
            ).wait()

    return gather_kernel


_TR_BLK = 2048


def _make_relayout(vocab: int, dim: int):
    grid = (vocab + _TR_BLK - 1) // _TR_BLK

    def body(t_ref, o_ref):
        eye = jnp.eye(dim, dtype=jnp.float32)
        o_ref[:, :dim] = jax.lax.dot_general(
            t_ref[...], eye, (((0,), (0,)), ((), ())),
            preferred_element_type=jnp.float32,
            precision=jax.lax.Precision.HIGHEST,
        )

    return pl.pallas_call(
        body,
        grid=(grid,),
        in_specs=[pl.BlockSpec((dim, _TR_BLK), lambda i: (0, i))],
        out_specs=pl.BlockSpec((_TR_BLK, 2 * dim), lambda i: (i, 0)),
        out_shape=jax.ShapeDtypeStruct((vocab, 2 * dim), jnp.float32),
    )


def kernel(idxes, table):
    batch, seq = idxes.shape
    vocab, dim = table.shape
    table2 = _make_relayout(vocab, dim)(table.T).reshape(2 * vocab, dim)
    out = _make_gather(batch, seq, dim)(idxes * 2, table2)
    return out.reshape(batch, seq, 2 * dim)[:, :, :dim]


# speedup vs baseline: 1.1528x; 1.1528x over previous
import functools

import jax
import jax.numpy as jnp
from jax import lax
from jax.experimental import pallas as pl
from jax.experimental.pallas import tpu as pltpu
from jax.experimental.pallas import tpu_sc as plsc

_INFO = plsc.get_sparse_core_info()
_NC = _INFO.num_cores
_NS = _INFO.num_subcores
_NW = _NC * _NS

_CB = 4
_NBUF = 2


def _make_gather(batch: int, seq: int, dim: int):
    assert batch % (_NW * _CB * _NBUF) == 0
    rows_per_w = batch // _NW
    n_groups = rows_per_w // (_CB * _NBUF)
    chunk = _CB * seq
    mesh = plsc.VectorSubcoreMesh(core_axis_name="c", subcore_axis_name="s")

    @functools.partial(
        pl.kernel,
        mesh=mesh,
        out_type=jax.ShapeDtypeStruct((batch * seq, 2 * dim), jnp.float32),
        scratch_types=[
            [pltpu.VMEM((_CB, seq), jnp.int32) for _ in range(_NBUF)],
            [pltpu.VMEM((chunk, dim), jnp.float32) for _ in range(_NBUF)],
            [pltpu.SemaphoreType.DMA for _ in range(_NBUF)],
            [pltpu.SemaphoreType.DMA for _ in range(_NBUF)],
        ],
        compiler_params=pltpu.CompilerParams(use_tc_tiling_on_sc=False),
    )
    def gather_kernel(idx_hbm, table_hbm, out_hbm, idx_v, rows, gsems, wsems):
        wid = lax.axis_index("s") * _NC + lax.axis_index("c")
        base_row = wid * rows_per_w

        def group(g, carry):
            gathers = []
            for b in range(_NBUF):
                r0 = base_row + (g * _NBUF + b) * _CB

                @pl.when(g > 0)
                def _(b=b, r0=r0):
                    pltpu.make_async_copy(
                        rows[b],
                        out_hbm.at[pl.ds(r0 * seq, chunk), pl.ds(0, dim)],
                        wsems[b],
                    ).wait()

                pltpu.sync_copy(idx_hbm.at[pl.ds(r0, _CB)], idx_v[b])
                for j in range(_CB):
                    gathers.append(
                        pltpu.async_copy(
                            table_hbm.at[idx_v[b].at[j]],
                            rows[b].at[pl.ds(j * seq, seq)],
                            gsems[b],
                        )
                    )
            for b in range(_NBUF):
                r0 = base_row + (g * _NBUF + b) * _CB
                for j in range(_CB):
                    gathers[b * _CB + j].wait()
                pltpu.async_copy(
                    rows[b],
                    out_hbm.at[pl.ds(r0 * seq, chunk), pl.ds(0, dim)],
                    wsems[b],
                )
            return carry

        lax.fori_loop(0, n_groups, group, 0)

        for b in range(_NBUF):
            r0 = base_row + ((n_groups - 1) * _NBUF + b) * _CB
            pltpu.make_async_copy(
                rows[b],
                out_hbm.at[pl.ds(r0 * seq, chunk), pl.ds(0, dim)],
                wsems[b],
            ).wait()

    return gather_kernel


_TR_BLK = 2048


def _make_relayout(vocab: int, dim: int):
    grid = (vocab + _TR_BLK - 1) // _TR_BLK

    def body(t_ref, o_ref):
        eye = jnp.eye(dim, dtype=jnp.float32)
        o_ref[:, :dim] = jax.lax.dot_general(
            t_ref[...], eye, (((0,), (0,)), ((), ())),
            preferred_element_type=jnp.float32,
        )

    return pl.pallas_call(
        body,
        grid=(grid,),
        in_specs=[pl.BlockSpec((dim, _TR_BLK), lambda i: (0, i))],
        out_specs=pl.BlockSpec((_TR_BLK, 2 * dim), lambda i: (i, 0)),
        out_shape=jax.ShapeDtypeStruct((vocab, 2 * dim), jnp.float32),
    )


def kernel(idxes, table):
    batch, seq = idxes.shape
    vocab, dim = table.shape
    table2 = _make_relayout(vocab, dim)(table.T).reshape(2 * vocab, dim)
    out = _make_gather(batch, seq, dim)(idxes * 2, table2)
    return out.reshape(batch, seq, 2 * dim)[:, :, :dim]


# speedup vs baseline: 1.1827x; 1.0259x over previous
import functools

import jax
import jax.numpy as jnp
from jax import lax
from jax.experimental import pallas as pl
from jax.experimental.pallas import tpu as pltpu
from jax.experimental.pallas import tpu_sc as plsc

_INFO = plsc.get_sparse_core_info()
_NC = _INFO.num_cores
_NS = _INFO.num_subcores
_NW = _NC * _NS

_CB = 4
_NBUF = 2


def _make_gather(batch: int, seq: int, dim: int):
    assert batch % (_NW * _CB * _NBUF) == 0
    rows_per_w = batch // _NW
    n_groups = rows_per_w // (_CB * _NBUF)
    chunk = _CB * seq
    mesh = plsc.VectorSubcoreMesh(core_axis_name="c", subcore_axis_name="s")

    @functools.partial(
        pl.kernel,
        mesh=mesh,
        out_type=jax.ShapeDtypeStruct((batch * seq, 2 * dim), jnp.float32),
        scratch_types=[
            [pltpu.VMEM((_CB, seq), jnp.int32) for _ in range(_NBUF)],
            [pltpu.VMEM((chunk, dim), jnp.float32) for _ in range(_NBUF)],
            [pltpu.SemaphoreType.DMA for _ in range(_NBUF)],
            [pltpu.SemaphoreType.DMA for _ in range(_NBUF)],
        ],
        compiler_params=pltpu.CompilerParams(use_tc_tiling_on_sc=False),
    )
    def gather_kernel(idx_hbm, table_hbm, out_hbm, idx_v, rows, gsems, wsems):
        wid = lax.axis_index("s") * _NC + lax.axis_index("c")
        base_row = wid * rows_per_w

        def group(g, carry):
            gathers = []
            for b in range(_NBUF):
                r0 = base_row + (g * _NBUF + b) * _CB

                @pl.when(g > 0)
                def _(b=b, r0=r0):
                    pltpu.make_async_copy(
                        rows[b],
                        out_hbm.at[pl.ds(r0 * seq, chunk), pl.ds(0, dim)],
                        wsems[b],
                    ).wait()

                pltpu.sync_copy(idx_hbm.at[pl.ds(r0, _CB)], idx_v[b])
                for j in range(_CB):
                    gathers.append(
                        pltpu.async_copy(
                            table_hbm.at[idx_v[b].at[j]],
                            rows[b].at[pl.ds(j * seq, seq)],
                            gsems[b],
                        )
                    )
            for b in range(_NBUF):
                r0 = base_row + (g * _NBUF + b) * _CB
                for j in range(_CB):
                    gathers[b * _CB + j].wait()
                pltpu.async_copy(
                    rows[b],
                    out_hbm.at[pl.ds(r0 * seq, chunk), pl.ds(0, dim)],
                    wsems[b],
                )
            return carry

        lax.fori_loop(0, n_groups, group, 0)

        for b in range(_NBUF):
            r0 = base_row + ((n_groups - 1) * _NBUF + b) * _CB
            pltpu.make_async_copy(
                rows[b],
                out_hbm.at[pl.ds(r0 * seq, chunk), pl.ds(0, dim)],
                wsems[b],
            ).wait()

    return gather_kernel


_TR_BLK = 2048


def _make_relayout(vocab: int, dim: int):
    grid = (vocab + _TR_BLK - 1) // _TR_BLK

    def body(t_ref, o_ref):
        o_ref[:, :dim] = t_ref[...].T

    return pl.pallas_call(
        body,
        grid=(grid,),
        in_specs=[pl.BlockSpec((dim, _TR_BLK), lambda i: (0, i))],
        out_specs=pl.BlockSpec((_TR_BLK, 2 * dim), lambda i: (i, 0)),
        out_shape=jax.ShapeDtypeStruct((vocab, 2 * dim), jnp.float32),
    )


def kernel(idxes, table):
    batch, seq = idxes.shape
    vocab, dim = table.shape
    table2 = _make_relayout(vocab, dim)(table.T).reshape(2 * vocab, dim)
    out = _make_gather(batch, seq, dim)(idxes * 2, table2)
    return out.reshape(batch, seq, 2 * dim)[:, :, :dim]


# speedup vs baseline: 1.5699x; 1.3274x over previous
import functools

import jax
import jax.numpy as jnp
from jax import lax
from jax.experimental import pallas as pl
from jax.experimental.pallas import tpu as pltpu
from jax.experimental.pallas import tpu_sc as plsc

_INFO = plsc.get_sparse_core_info()
_NC = _INFO.num_cores
_NS = _INFO.num_subcores
_NW = _NC * _NS

_CB = 4
_NBUF = 2


def _make_gather(batch: int, seq: int, dim: int):
    assert batch % (_NW * _CB * _NBUF) == 0
    rows_per_w = batch // _NW
    n_groups = rows_per_w // (_CB * _NBUF)
    chunk = _CB * seq
    mesh = plsc.VectorSubcoreMesh(core_axis_name="c", subcore_axis_name="s")

    @functools.partial(
        pl.kernel,
        mesh=mesh,
        out_type=jax.ShapeDtypeStruct((batch * seq, 2 * dim), jnp.float32),
        scratch_types=[
            [pltpu.VMEM((_CB, seq), jnp.int32) for _ in range(_NBUF)],
            [pltpu.VMEM((chunk, dim), jnp.float32) for _ in range(_NBUF)],
            [pltpu.SemaphoreType.DMA for _ in range(_NBUF)],
            [pltpu.SemaphoreType.DMA for _ in range(_NBUF)],
        ],
        compiler_params=pltpu.CompilerParams(use_tc_tiling_on_sc=False),
    )
    def gather_kernel(idx_hbm, table_hbm, out_hbm, idx_v, rows, gsems, wsems):
        wid = lax.axis_index("s") * _NC + lax.axis_index("c")
        base_row = wid * rows_per_w

        def group(g, carry):
            gathers = []
            for b in range(_NBUF):
                r0 = base_row + (g * _NBUF + b) * _CB

                @pl.when(g > 0)
                def _(b=b, r0=r0):
                    pltpu.make_async_copy(
                        rows[b],
                        out_hbm.at[pl.ds(r0 * seq, chunk), pl.ds(0, dim)],
                        wsems[b],
                    ).wait()

                pltpu.sync_copy(idx_hbm.at[pl.ds(r0, _CB)], idx_v[b])
                for j in range(_CB):
                    gathers.append(
                        pltpu.async_copy(
                            table_hbm.at[idx_v[b].at[j]],
                            rows[b].at[pl.ds(j * seq, seq)],
                            gsems[b],
                        )
                    )
            for b in range(_NBUF):
                r0 = base_row + (g * _NBUF + b) * _CB
                for j in range(_CB):
                    gathers[b * _CB + j].wait()
                pltpu.async_copy(
                    rows[b],
                    out_hbm.at[pl.ds(r0 * seq, chunk), pl.ds(0, dim)],
                    wsems[b],
                )
            return carry

        lax.fori_loop(0, n_groups, group, 0)

        for b in range(_NBUF):
            r0 = base_row + ((n_groups - 1) * _NBUF + b) * _CB
            pltpu.make_async_copy(
                rows[b],
                out_hbm.at[pl.ds(r0 * seq, chunk), pl.ds(0, dim)],
                wsems[b],
            ).wait()

    return gather_kernel


_TR_BLK = 8192


def _make_relayout(vocab: int, dim: int):
    grid = (vocab + _TR_BLK - 1) // _TR_BLK

    def body(t_ref, o_ref):
        o_ref[:, :dim] = t_ref[...].T

    return pl.pallas_call(
        body,
        grid=(grid,),
        in_specs=[pl.BlockSpec((dim, _TR_BLK), lambda i: (0, i))],
        out_specs=pl.BlockSpec((_TR_BLK, 2 * dim), lambda i: (i, 0)),
        out_shape=jax.ShapeDtypeStruct((vocab, 2 * dim), jnp.float32),
    )


def kernel(idxes, table):
    batch, seq = idxes.shape
    vocab, dim = table.shape
    table2 = _make_relayout(vocab, dim)(table.T).reshape(2 * vocab, dim)
    out = _make_gather(batch, seq, dim)(idxes * 2, table2)
    return out.reshape(batch, seq, 2 * dim)[:, :, :dim]


# speedup vs baseline: 1.6180x; 1.0306x over previous
import functools

import jax
import jax.numpy as jnp
from jax import lax
from jax.experimental import pallas as pl
from jax.experimental.pallas import tpu as pltpu
from jax.experimental.pallas import tpu_sc as plsc

_INFO = plsc.get_sparse_core_info()
_NC = _INFO.num_cores
_NS = _INFO.num_subcores
_NW = _NC * _NS

_CB = 4
_NBUF = 2


def _make_gather(batch: int, seq: int, dim: int):
    assert batch % (_NW * _CB * _NBUF) == 0
    rows_per_w = batch // _NW
    n_groups = rows_per_w // (_CB * _NBUF)
    chunk = _CB * seq
    mesh = plsc.VectorSubcoreMesh(core_axis_name="c", subcore_axis_name="s")

    @functools.partial(
        pl.kernel,
        mesh=mesh,
        out_type=jax.ShapeDtypeStruct((batch * seq, 2 * dim), jnp.float32),
        scratch_types=[
            [pltpu.VMEM((_CB, seq), jnp.int32) for _ in range(_NBUF)],
            [pltpu.VMEM((chunk, dim), jnp.float32) for _ in range(_NBUF)],
            [pltpu.SemaphoreType.DMA for _ in range(_NBUF)],
            [pltpu.SemaphoreType.DMA for _ in range(_NBUF)],
        ],
        compiler_params=pltpu.CompilerParams(use_tc_tiling_on_sc=False),
    )
    def gather_kernel(idx_hbm, table_hbm, out_hbm, idx_v, rows, gsems, wsems):
        wid = lax.axis_index("s") * _NC + lax.axis_index("c")
        base_row = wid * rows_per_w

        def group(g, carry):
            gathers = []
            for b in range(_NBUF):
                r0 = base_row + (g * _NBUF + b) * _CB

                @pl.when(g > 0)
                def _(b=b, r0=r0):
                    pltpu.make_async_copy(
                        rows[b],
                        out_hbm.at[pl.ds(r0 * seq, chunk), pl.ds(0, dim)],
                        wsems[b],
                    ).wait()

                pltpu.sync_copy(idx_hbm.at[pl.ds(r0, _CB)], idx_v[b])
                for j in range(_CB):
                    gathers.append(
                        pltpu.async_copy(
                            table_hbm.at[idx_v[b].at[j]],
                            rows[b].at[pl.ds(j * seq, seq)],
                            gsems[b],
                        )
                    )
            for b in range(_NBUF):
                r0 = base_row + (g * _NBUF + b) * _CB
                for j in range(_CB):
                    gathers[b * _CB + j].wait()
                pltpu.async_copy(
                    rows[b],
                    out_hbm.at[pl.ds(r0 * seq, chunk), pl.ds(0, dim)],
                    wsems[b],
                )
            return carry

        lax.fori_loop(0, n_groups, group, 0)

        for b in range(_NBUF):
            r0 = base_row + ((n_groups - 1) * _NBUF + b) * _CB
            pltpu.make_async_copy(
                rows[b],
                out_hbm.at[pl.ds(r0 * seq, chunk), pl.ds(0, dim)],
                wsems[b],
            ).wait()

    return gather_kernel


_TR_BLK = 16384


def _make_relayout(vocab: int, dim: int):
    grid = (vocab + _TR_BLK - 1) // _TR_BLK

    def body(t_ref, o_ref):
        o_ref[:, :dim] = t_ref[...].T

    return pl.pallas_call(
        body,
        grid=(grid,),
        in_specs=[pl.BlockSpec((dim, _TR_BLK), lambda i: (0, i))],
        out_specs=pl.BlockSpec((_TR_BLK, 2 * dim), lambda i: (i, 0)),
        out_shape=jax.ShapeDtypeStruct((vocab, 2 * dim), jnp.float32),
    )


def kernel(idxes, table):
    batch, seq = idxes.shape
    vocab, dim = table.shape
    table2 = _make_relayout(vocab, dim)(table.T).reshape(2 * vocab, dim)
    out = _make_gather(batch, seq, dim)(idxes * 2, table2)
    return out.reshape(batch, seq, 2 * dim)[:, :, :dim]


# speedup vs baseline: 1.6336x; 1.0096x over previous
import functools

import jax
import jax.numpy as jnp
from jax import lax
from jax.experimental import pallas as pl
from jax.experimental.pallas import tpu as pltpu
from jax.experimental.pallas import tpu_sc as plsc

_INFO = plsc.get_sparse_core_info()
_NC = _INFO.num_cores
_NS = _INFO.num_subcores
_NW = _NC * _NS

_CB = 4
_NBUF = 2


def _make_gather(batch: int, seq: int, dim: int):
    assert batch % (_NW * _CB * _NBUF) == 0
    rows_per_w = batch // _NW
    n_groups = rows_per_w // (_CB * _NBUF)
    chunk = _CB * seq
    mesh = plsc.VectorSubcoreMesh(core_axis_name="c", subcore_axis_name="s")

    @functools.partial(
        pl.kernel,
        mesh=mesh,
        out_type=jax.ShapeDtypeStruct((batch * seq, 2 * dim), jnp.float32),
        scratch_types=[
            [pltpu.VMEM((_CB, seq), jnp.int32) for _ in range(_NBUF)],
            [pltpu.VMEM((chunk, dim), jnp.float32) for _ in range(_NBUF)],
            [pltpu.SemaphoreType.DMA for _ in range(_NBUF)],
            [pltpu.SemaphoreType.DMA for _ in range(_NBUF)],
        ],
        compiler_params=pltpu.CompilerParams(use_tc_tiling_on_sc=False),
    )
    def gather_kernel(idx_hbm, table_hbm, out_hbm, idx_v, rows, gsems, wsems):
        wid = lax.axis_index("s") * _NC + lax.axis_index("c")
        base_row = wid * rows_per_w

        def group(g, carry):
            gathers = []
            for b in range(_NBUF):
                r0 = base_row + (g * _NBUF + b) * _CB

                @pl.when(g > 0)
                def _(b=b, r0=r0):
                    pltpu.make_async_copy(
                        rows[b],
                        out_hbm.at[pl.ds(r0 * seq, chunk), pl.ds(0, dim)],
                        wsems[b],
                    ).wait()

                pltpu.sync_copy(idx_hbm.at[pl.ds(r0, _CB)], idx_v[b])
                for j in range(_CB):
                    gathers.append(
                        pltpu.async_copy(
                            table_hbm.at[idx_v[b].at[j]],
                            rows[b].at[pl.ds(j * seq, seq)],
                            gsems[b],
                        )
                    )
            for b in range(_NBUF):
                r0 = base_row + (g * _NBUF + b) * _CB
                for j in range(_CB):
                    gathers[b * _CB + j].wait()
                pltpu.async_copy(
                    rows[b],
                    out_hbm.at[pl.ds(r0 * seq, chunk), pl.ds(0, dim)],
                    wsems[b],
                )
            return carry

        lax.fori_loop(0, n_groups, group, 0)

        for b in range(_NBUF):
            r0 = base_row + ((n_groups - 1) * _NBUF + b) * _CB
            pltpu.make_async_copy(
                rows[b],
                out_hbm.at[pl.ds(r0 * seq, chunk), pl.ds(0, dim)],
                wsems[b],
            ).wait()

    return gather_kernel


_TR_BLK = 32768


def _make_relayout(vocab: int, dim: int):
    grid = (vocab + _TR_BLK - 1) // _TR_BLK

    def body(t_ref, o_ref):
        o_ref[:, :dim] = t_ref[...].T

    return pl.pallas_call(
        body,
        grid=(grid,),
        in_specs=[pl.BlockSpec((dim, _TR_BLK), lambda i: (0, i))],
        out_specs=pl.BlockSpec((_TR_BLK, 2 * dim), lambda i: (i, 0)),
        out_shape=jax.ShapeDtypeStruct((vocab, 2 * dim), jnp.float32),
    )


def kernel(idxes, table):
    batch, seq = idxes.shape
    vocab, dim = table.shape
    table2 = _make_relayout(vocab, dim)(table.T).reshape(2 * vocab, dim)
    out = _make_gather(batch, seq, dim)(idxes * 2, table2)
    return out.reshape(batch, seq, 2 * dim)[:, :, :dim]


# speedup vs baseline: 1.6372x; 1.0022x over previous
import functools

import jax
import jax.numpy as jnp
from jax import lax
from jax.experimental import pallas as pl
from jax.experimental.pallas import tpu as pltpu
from jax.experimental.pallas import tpu_sc as plsc

_INFO = plsc.get_sparse_core_info()
_NC = _INFO.num_cores
_NS = _INFO.num_subcores
_NW = _NC * _NS

_CB = 2
_NBUF = 4


def _make_gather(batch: int, seq: int, dim: int):
    assert batch % (_NW * _CB * _NBUF) == 0
    rows_per_w = batch // _NW
    n_groups = rows_per_w // (_CB * _NBUF)
    chunk = _CB * seq
    mesh = plsc.VectorSubcoreMesh(core_axis_name="c", subcore_axis_name="s")

    @functools.partial(
        pl.kernel,
        mesh=mesh,
        out_type=jax.ShapeDtypeStruct((batch * seq, 2 * dim), jnp.float32),
        scratch_types=[
            [pltpu.VMEM((_CB, seq), jnp.int32) for _ in range(_NBUF)],
            [pltpu.VMEM((chunk, dim), jnp.float32) for _ in range(_NBUF)],
            [pltpu.SemaphoreType.DMA for _ in range(_NBUF)],
            [pltpu.SemaphoreType.DMA for _ in range(_NBUF)],
        ],
        compiler_params=pltpu.CompilerParams(use_tc_tiling_on_sc=False),
    )
    def gather_kernel(idx_hbm, table_hbm, out_hbm, idx_v, rows, gsems, wsems):
        wid = lax.axis_index("s") * _NC + lax.axis_index("c")
        base_row = wid * rows_per_w

        def group(g, carry):
            gathers = []
            for b in range(_NBUF):
                r0 = base_row + (g * _NBUF + b) * _CB

                @pl.when(g > 0)
                def _(b=b, r0=r0):
                    pltpu.make_async_copy(
                        rows[b],
                        out_hbm.at[pl.ds(r0 * seq, chunk), pl.ds(0, dim)],
                        wsems[b],
                    ).wait()

                pltpu.sync_copy(idx_hbm.at[pl.ds(r0, _CB)], idx_v[b])
                for j in range(_CB):
                    gathers.append(
                        pltpu.async_copy(
                            table_hbm.at[idx_v[b].at[j]],
                            rows[b].at[pl.ds(j * seq, seq)],
                            gsems[b],
                        )
                    )
            for b in range(_NBUF):
                r0 = base_row + (g * _NBUF + b) * _CB
                for j in range(_CB):
                    gathers[b * _CB + j].wait()
                pltpu.async_copy(
                    rows[b],
                    out_hbm.at[pl.ds(r0 * seq, chunk), pl.ds(0, dim)],
                    wsems[b],
                )
            return carry

        lax.fori_loop(0, n_groups, group, 0)

        for b in range(_NBUF):
            r0 = base_row + ((n_groups - 1) * _NBUF + b) * _CB
            pltpu.make_async_copy(
                rows[b],
                out_hbm.at[pl.ds(r0 * seq, chunk), pl.ds(0, dim)],
                wsems[b],
            ).wait()

    return gather_kernel


_TR_BLK = 32768


def _make_relayout(vocab: int, dim: int):
    grid = (vocab + _TR_BLK - 1) // _TR_BLK

    def body(t_ref, o_ref):
        o_ref[:, :dim] = t_ref[...].T

    return pl.pallas_call(
        body,
        grid=(grid,),
        in_specs=[pl.BlockSpec((dim, _TR_BLK), lambda i: (0, i))],
        out_specs=pl.BlockSpec((_TR_BLK, 2 * dim), lambda i: (i, 0)),
        out_shape=jax.ShapeDtypeStruct((vocab, 2 * dim), jnp.float32),
    )


def kernel(idxes, table):
    batch, seq = idxes.shape
    vocab, dim = table.shape
    table2 = _make_relayout(vocab, dim)(table.T).reshape(2 * vocab, dim)
    out = _make_gather(batch, seq, dim)(idxes * 2, table2)
    return out.reshape(batch, seq, 2 * dim)[:, :, :dim]
